# baseline (speedup 1.0000x reference)
import functools

import jax
import jax.numpy as jnp
from jax import lax
from jax.experimental import pallas as pl
from jax.experimental.pallas import tpu as pltpu
from jax.experimental.pallas import tpu_sc as plsc

KB = 8192


def _argmin_body(n_keys, q_ref, k_ref, out_ref, best_val, best_idx):
    pid = pl.program_id(0)
    nb = pl.num_programs(0)

    @pl.when(pid == 0)
    def _init():
        best_val[...] = jnp.full_like(best_val, jnp.inf)
        best_idx[...] = jnp.zeros_like(best_idx)

    kb = k_ref[...]
    qk = lax.dot_general(q_ref[...], kb, (((1,), (1,)), ((), ())),
                         preferred_element_type=jnp.float32)
    ones = jnp.ones((1, kb.shape[1]), jnp.float32)
    ksq = lax.dot_general(ones, kb * kb, (((1,), (1,)), ((), ())),
                          preferred_element_type=jnp.float32)
    scores = ksq - 2.0 * qk
    gidx = pid * KB + lax.broadcasted_iota(jnp.int32, scores.shape, 1)

    scores = jnp.where((pid != nb - 1) | (gidx < n_keys), scores, jnp.inf)
    local_min = jnp.min(scores, axis=1, keepdims=True)
    big = jnp.int32(jnp.iinfo(jnp.int32).max)
    local_arg = jnp.min(jnp.where(scores == local_min, gidx, big),
                        axis=1, keepdims=True)
    improved = local_min < best_val[...]
    best_val[...] = jnp.where(improved, local_min, best_val[...])
    best_idx[...] = jnp.where(improved, local_arg, best_idx[...])

    @pl.when(pid == nb - 1)
    def _fin():
        out_ref[...] = best_idx[...]


def _nearest_tc(queries, keys):
    q, d = queries.shape
    n_keys = keys.shape[0]
    grid = (n_keys + KB - 1) // KB
    return pl.pallas_call(
        functools.partial(_argmin_body, n_keys),
        grid=(grid,),
        in_specs=[
            pl.BlockSpec((q, d), lambda i: (0, 0)),
            pl.BlockSpec((KB, d), lambda i: (i, 0)),
        ],
        out_specs=pl.BlockSpec((q, 1), lambda i: (0, 0)),
        out_shape=jax.ShapeDtypeStruct((q, 1), jnp.int32),
        scratch_shapes=[
            pltpu.VMEM((q, 1), jnp.float32),
            pltpu.VMEM((q, 1), jnp.int32),
        ],
    )(queries, keys)


def _sc_gather_body(q, dc, nearest_hbm, state_hbm, ccflat_hbm, out_hbm,
                    idx_v, sidx_v, eidx_v, rows_v, sem):
    wid = lax.axis_index("s") * 2 + lax.axis_index("c")
    dc_shift = dc.bit_length() - 1

    @pl.when(wid == 0)
    def _():
        pltpu.sync_copy(nearest_hbm, idx_v)
        pltpu.async_copy(state_hbm.at[idx_v], sidx_v, sem).wait()
        for i in range(q // 16):
            base = lax.shift_left(sidx_v[pl.ds(i * 16, 16)], dc_shift)
            for j in range(dc):
                eidx_v[j, pl.ds(i * 16, 16)] = base + j
        cps = [pltpu.async_copy(ccflat_hbm.at[eidx_v.at[j]], rows_v.at[j], sem)
               for j in range(dc)]
        for cp in cps:
            cp.wait()
        pltpu.sync_copy(rows_v, out_hbm)


def _centers_sc(nearest, state_indices, cc_flat, dc):
    q = nearest.shape[0]
    mesh = plsc.VectorSubcoreMesh(core_axis_name="c", subcore_axis_name="s")
    return pl.kernel(
        functools.partial(_sc_gather_body, q, dc),
        out_type=jax.ShapeDtypeStruct((dc, q), jnp.float32),
        mesh=mesh,
        scratch_types=[
            pltpu.VMEM((q,), jnp.int32),
            pltpu.VMEM((q,), jnp.int32),
            pltpu.VMEM((dc, q), jnp.int32),
            pltpu.VMEM((dc, q), jnp.float32),
            pltpu.SemaphoreType.DMA,
        ],
    )(nearest, state_indices, cc_flat)


def kernel(queries, keys, cluster_centers, state_indices):
    q = queries.shape[0]
    dc = cluster_centers.shape[1]
    nearest = _nearest_tc(queries, keys).reshape((q,))
    comp = _centers_sc(nearest, state_indices, cluster_centers.reshape((-1,)), dc)
    return comp.T


# speedup vs baseline: 1.0257x; 1.0257x over previous
import functools

import jax
import jax.numpy as jnp
from jax import lax
from jax.experimental import pallas as pl
from jax.experimental.pallas import tpu as pltpu
from jax.experimental.pallas import tpu_sc as plsc

KB = 16384
GROUP = 128


def _argmin_body(n_keys, q_ref, k_ref, out_ref, best_val, best_idx):
    pid = pl.program_id(0)
    nb = pl.num_programs(0)

    @pl.when(pid == 0)
    def _init():
        best_val[...] = jnp.full_like(best_val, jnp.inf)
        best_idx[...] = jnp.zeros_like(best_idx)

    kb = k_ref[...]
    qk = lax.dot_general(q_ref[...], kb, (((1,), (1,)), ((), ())),
                         preferred_element_type=jnp.float32)
    ones = jnp.ones((1, kb.shape[1]), jnp.float32)
    ksq = lax.dot_general(ones, kb * kb, (((1,), (1,)), ((), ())),
                          preferred_element_type=jnp.float32)
    scores = ksq - 2.0 * qk
    lidx = lax.broadcasted_iota(jnp.int32, scores.shape, 1)
    scores = jnp.where((pid != nb - 1) | (lidx < (n_keys - pid * KB)),
                       scores, jnp.inf)
    local_min = jnp.min(scores, axis=1, keepdims=True)
    big = jnp.int32(jnp.iinfo(jnp.int32).max)
    local_arg = jnp.min(jnp.where(scores == local_min, lidx, big),
                        axis=1, keepdims=True)
    improved = local_min < best_val[...]
    best_val[...] = jnp.where(improved, local_min, best_val[...])
    best_idx[...] = jnp.where(improved, pid * KB + local_arg, best_idx[...])

    @pl.when(pid == nb - 1)
    def _fin():
        out_ref[...] = best_idx[...]


def _nearest_tc(queries, keys):
    q, d = queries.shape
    n_keys = keys.shape[0]
    grid = (n_keys + KB - 1) // KB
    return pl.pallas_call(
        functools.partial(_argmin_body, n_keys),
        grid=(grid,),
        in_specs=[
            pl.BlockSpec((q, d), lambda i: (0, 0)),
            pl.BlockSpec((KB, d), lambda i: (i, 0)),
        ],
        out_specs=pl.BlockSpec((q, 1), lambda i: (0, 0)),
        out_shape=jax.ShapeDtypeStruct((q, 1), jnp.int32),
        scratch_shapes=[
            pltpu.VMEM((q, 1), jnp.float32),
            pltpu.VMEM((q, 1), jnp.int32),
        ],
    )(queries, keys)


def _sc_gather_body(dc, nearest_hbm, state_hbm, ccg_hbm,
                    sidx_out, grp_out, idx_v, sidx_v, gidx_v, grp_v, sem):
    wid = lax.axis_index("s") * 2 + lax.axis_index("c")
    rows_per_group = GROUP // dc
    g_shift = rows_per_group.bit_length() - 1

    @pl.when(wid == 0)
    def _():
        pltpu.sync_copy(nearest_hbm, idx_v)
        pltpu.async_copy(state_hbm.at[idx_v], sidx_v, sem).wait()
        q = idx_v.shape[0]
        for i in range(q // 16):
            gidx_v[pl.ds(i * 16, 16)] = lax.shift_right_logical(
                sidx_v[pl.ds(i * 16, 16)], g_shift)
        pltpu.async_copy(ccg_hbm.at[gidx_v], grp_v, sem).wait()
        pltpu.sync_copy(sidx_v, sidx_out)
        pltpu.sync_copy(grp_v, grp_out)


def _centers_sc(nearest, state_indices, cc_groups, dc):
    q = nearest.shape[0]
    mesh = plsc.VectorSubcoreMesh(core_axis_name="c", subcore_axis_name="s")
    return pl.kernel(
        functools.partial(_sc_gather_body, dc),
        out_type=(
            jax.ShapeDtypeStruct((q,), jnp.int32),
            jax.ShapeDtypeStruct((q, GROUP), jnp.float32),
        ),
        mesh=mesh,
        scratch_types=[
            pltpu.VMEM((q,), jnp.int32),
            pltpu.VMEM((q,), jnp.int32),
            pltpu.VMEM((q,), jnp.int32),
            pltpu.VMEM((q, GROUP), jnp.float32),
            pltpu.SemaphoreType.DMA,
        ],
    )(nearest, state_indices, cc_groups)


def kernel(queries, keys, cluster_centers, state_indices):
    q = queries.shape[0]
    dc = cluster_centers.shape[1]
    nearest = _nearest_tc(queries, keys).reshape((q,))
    cc_groups = cluster_centers.reshape((-1, GROUP))
    sidx, groups = _centers_sc(nearest, state_indices, cc_groups, dc)
    off = (sidx % (GROUP // dc))[:, None] * dc + jnp.arange(dc)[None, :]
    return jnp.take_along_axis(groups, off, axis=1)


# speedup vs baseline: 1.1622x; 1.1331x over previous
import functools

import jax
import jax.numpy as jnp
from jax import lax
from jax.experimental import pallas as pl
from jax.experimental.pallas import tpu as pltpu
from jax.experimental.pallas import tpu_sc as plsc

KB = 32768


def _argmin_body(n_keys, q_ref, k_ref, out_ref, best_val, best_idx):
    pid = pl.program_id(0)
    nb = pl.num_programs(0)

    @pl.when(pid == 0)
    def _init():
        best_val[...] = jnp.full_like(best_val, jnp.inf)
        best_idx[...] = jnp.zeros_like(best_idx)

    kb = k_ref[...]
    qk = lax.dot_general(q_ref[...], kb, (((1,), (1,)), ((), ())),
                         preferred_element_type=jnp.float32)
    ones = jnp.ones((1, kb.shape[1]), jnp.float32)
    ksq = lax.dot_general(ones, kb * kb, (((1,), (1,)), ((), ())),
                          preferred_element_type=jnp.float32)
    scores = ksq - 2.0 * qk
    lidx = lax.broadcasted_iota(jnp.int32, scores.shape, 1)
    scores = jnp.where((pid != nb - 1) | (lidx < (n_keys - pid * KB)),
                       scores, jnp.inf)
    local_min = jnp.min(scores, axis=1, keepdims=True)
    big = jnp.int32(jnp.iinfo(jnp.int32).max)
    local_arg = jnp.min(jnp.where(scores == local_min, lidx, big),
                        axis=1, keepdims=True)
    improved = local_min < best_val[...]
    best_val[...] = jnp.where(improved, local_min, best_val[...])
    best_idx[...] = jnp.where(improved, pid * KB + local_arg, best_idx[...])

    @pl.when(pid == nb - 1)
    def _fin():
        out_ref[...] = best_idx[...]


def _nearest_tc(queries, keys):
    q, d = queries.shape
    n_keys = keys.shape[0]
    grid = (n_keys + KB - 1) // KB
    return pl.pallas_call(
        functools.partial(_argmin_body, n_keys),
        grid=(grid,),
        in_specs=[
            pl.BlockSpec((q, d), lambda i: (0, 0)),
            pl.BlockSpec((KB, d), lambda i: (i, 0)),
        ],
        out_specs=pl.BlockSpec((q, 1), lambda i: (0, 0)),
        out_shape=jax.ShapeDtypeStruct((q, 1), jnp.int32),
        scratch_shapes=[
            pltpu.VMEM((q, 1), jnp.float32),
            pltpu.VMEM((q, 1), jnp.int32),
        ],
    )(queries, keys)


def _sc_chase_body(nearest_hbm, state_hbm, sidx_out, idx_v, sidx_v, sem):
    wid = lax.axis_index("s") * 2 + lax.axis_index("c")

    @pl.when(wid == 0)
    def _():
        pltpu.sync_copy(nearest_hbm, idx_v)
        pltpu.async_copy(state_hbm.at[idx_v], sidx_v, sem).wait()
        pltpu.sync_copy(sidx_v, sidx_out)


def _chase_sc(nearest, state_indices):
    q = nearest.shape[0]
    mesh = plsc.VectorSubcoreMesh(core_axis_name="c", subcore_axis_name="s")
    return pl.kernel(
        _sc_chase_body,
        out_type=jax.ShapeDtypeStruct((q,), jnp.int32),
        mesh=mesh,
        scratch_types=[
            pltpu.VMEM((q,), jnp.int32),
            pltpu.VMEM((q,), jnp.int32),
            pltpu.SemaphoreType.DMA,
        ],
    )(nearest, state_indices)


def _row_gather_body(sidx_ref, cc_ref, out_ref):
    i = pl.program_id(0)
    r = sidx_ref[i] % 8
    out_ref[pl.ds(i % 8, 1), :] = cc_ref[pl.ds(r, 1), :]


def _gather_tc(cluster_centers, sidx):
    q = sidx.shape[0]
    dc = cluster_centers.shape[1]
    grid_spec = pltpu.PrefetchScalarGridSpec(
        num_scalar_prefetch=1,
        grid=(q,),
        in_specs=[pl.BlockSpec((8, dc), lambda i, s: (s[i] // 8, 0))],
        out_specs=pl.BlockSpec((8, dc), lambda i, s: (i // 8, 0)),
    )
    return pl.pallas_call(
        _row_gather_body,
        grid_spec=grid_spec,
        out_shape=jax.ShapeDtypeStruct((q, dc), jnp.float32),
    )(sidx, cluster_centers)


def kernel(queries, keys, cluster_centers, state_indices):
    q = queries.shape[0]
    nearest = _nearest_tc(queries, keys).reshape((q,))
    sidx = _chase_sc(nearest, state_indices)
    return _gather_tc(cluster_centers, sidx)


# speedup vs baseline: 1.1623x; 1.0000x over previous
import functools

import jax
import jax.numpy as jnp
from jax import lax
from jax.experimental import pallas as pl
from jax.experimental.pallas import tpu as pltpu
from jax.experimental.pallas import tpu_sc as plsc

KB = 32768


def _argmin_body(n_keys, q_ref, k_ref, out_ref, best_val, best_idx):
    pid = pl.program_id(0)
    nb = pl.num_programs(0)

    @pl.when(pid == 0)
    def _init():
        best_val[...] = jnp.full_like(best_val, jnp.inf)
        best_idx[...] = jnp.zeros_like(best_idx)

    kb = k_ref[...]
    qk = lax.dot_general(q_ref[...], kb, (((1,), (1,)), ((), ())),
                         preferred_element_type=jnp.float32)
    ones = jnp.ones((1, kb.shape[1]), jnp.float32)
    ksq = lax.dot_general(ones, kb * kb, (((1,), (1,)), ((), ())),
                          preferred_element_type=jnp.float32)
    scores = ksq + qk
    lidx = lax.broadcasted_iota(jnp.int32, scores.shape, 1)
    big = jnp.int32(jnp.iinfo(jnp.int32).max)

    def _update(s):
        local_min = jnp.min(s, axis=1, keepdims=True)
        local_arg = jnp.min(jnp.where(s == local_min, lidx, big),
                            axis=1, keepdims=True)
        improved = local_min < best_val[...]
        best_val[...] = jnp.where(improved, local_min, best_val[...])
        best_idx[...] = jnp.where(improved, pid * KB + local_arg,
                                  best_idx[...])

    @pl.when(pid < nb - 1)
    def _main():
        _update(scores)

    @pl.when(pid == nb - 1)
    def _fin():
        _update(jnp.where(lidx < (n_keys - pid * KB), scores, jnp.inf))
        out_ref[...] = best_idx[...]


def _nearest_tc(queries, keys):
    q, d = queries.shape
    n_keys = keys.shape[0]
    grid = (n_keys + KB - 1) // KB
    return pl.pallas_call(
        functools.partial(_argmin_body, n_keys),
        grid=(grid,),
        in_specs=[
            pl.BlockSpec((q, d), lambda i: (0, 0)),
            pl.BlockSpec((KB, d), lambda i: (i, 0)),
        ],
        out_specs=pl.BlockSpec((q, 1), lambda i: (0, 0)),
        out_shape=jax.ShapeDtypeStruct((q, 1), jnp.int32),
        scratch_shapes=[
            pltpu.VMEM((q, 1), jnp.float32),
            pltpu.VMEM((q, 1), jnp.int32),
        ],
    )(queries, keys)


def _sc_chase_body(nearest_hbm, state_hbm, sidx_out, idx_v, sidx_v, sem):
    wid = lax.axis_index("s") * 2 + lax.axis_index("c")

    @pl.when(wid == 0)
    def _():
        pltpu.sync_copy(nearest_hbm, idx_v)
        pltpu.async_copy(state_hbm.at[idx_v], sidx_v, sem).wait()
        pltpu.sync_copy(sidx_v, sidx_out)


def _chase_sc(nearest, state_indices):
    q = nearest.shape[0]
    mesh = plsc.VectorSubcoreMesh(core_axis_name="c", subcore_axis_name="s")
    return pl.kernel(
        _sc_chase_body,
        out_type=jax.ShapeDtypeStruct((q,), jnp.int32),
        mesh=mesh,
        scratch_types=[
            pltpu.VMEM((q,), jnp.int32),
            pltpu.VMEM((q,), jnp.int32),
            pltpu.SemaphoreType.DMA,
        ],
    )(nearest, state_indices)


def _row_gather_body(sidx_ref, cc_ref, out_ref):
    i = pl.program_id(0)
    r = sidx_ref[i] % 8
    out_ref[pl.ds(i % 8, 1), :] = cc_ref[pl.ds(r, 1), :]


def _gather_tc(cluster_centers, sidx):
    q = sidx.shape[0]
    dc = cluster_centers.shape[1]
    grid_spec = pltpu.PrefetchScalarGridSpec(
        num_scalar_prefetch=1,
        grid=(q,),
        in_specs=[pl.BlockSpec((8, dc), lambda i, s: (s[i] // 8, 0))],
        out_specs=pl.BlockSpec((8, dc), lambda i, s: (i // 8, 0)),
    )
    return pl.pallas_call(
        _row_gather_body,
        grid_spec=grid_spec,
        out_shape=jax.ShapeDtypeStruct((q, dc), jnp.float32),
    )(sidx, cluster_centers)


def kernel(queries, keys, cluster_centers, state_indices):
    q = queries.shape[0]
    nearest = _nearest_tc(queries * -2.0, keys).reshape((q,))
    sidx = _chase_sc(nearest, state_indices)
    return _gather_tc(cluster_centers, sidx)


# speedup vs baseline: 1.1683x; 1.0052x over previous
import functools

import jax
import jax.numpy as jnp
from jax import lax
from jax.experimental import pallas as pl
from jax.experimental.pallas import tpu as pltpu
from jax.experimental.pallas import tpu_sc as plsc

SB = 16384
NS = 2
KB = SB * NS


def _argmin_body(n_keys, q_ref, k0_ref, k1_ref, out_ref, best_val, best_idx):
    pid = pl.program_id(0)
    nb = pl.num_programs(0)

    @pl.when(pid == 0)
    def _init():
        best_val[...] = jnp.full_like(best_val, jnp.inf)
        best_idx[...] = jnp.zeros_like(best_idx)

    big = jnp.int32(jnp.iinfo(jnp.int32).max)

    def _scores(k_ref):
        kb = k_ref[...]
        qk = lax.dot_general(q_ref[...], kb, (((1,), (1,)), ((), ())),
                             preferred_element_type=jnp.float32)
        ones = jnp.ones((1, kb.shape[1]), jnp.float32)
        ksq = lax.dot_general(ones, kb * kb, (((1,), (1,)), ((), ())),
                              preferred_element_type=jnp.float32)
        return ksq + qk

    def _update(s, base):
        lidx = lax.broadcasted_iota(jnp.int32, s.shape, 1)
        local_min = jnp.min(s, axis=1, keepdims=True)
        local_arg = jnp.min(jnp.where(s == local_min, lidx, big),
                            axis=1, keepdims=True)
        improved = local_min < best_val[...]
        best_val[...] = jnp.where(improved, local_min, best_val[...])
        best_idx[...] = jnp.where(improved, base + local_arg,
                                  best_idx[...])

    def _masked(s, base):
        lidx = lax.broadcasted_iota(jnp.int32, s.shape, 1)
        return jnp.where(lidx < (n_keys - base), s, jnp.inf)

    s0 = _scores(k0_ref)
    s1 = _scores(k1_ref)
    b0 = pid * KB
    b1 = pid * KB + SB

    @pl.when(pid < nb - 1)
    def _main():
        _update(s0, b0)
        _update(s1, b1)

    @pl.when(pid == nb - 1)
    def _fin():
        _update(_masked(s0, b0), b0)
        _update(_masked(s1, b1), b1)
        out_ref[...] = best_idx[...]


def _nearest_tc(queries, keys):
    q, d = queries.shape
    n_keys = keys.shape[0]
    grid = (n_keys + KB - 1) // KB
    return pl.pallas_call(
        functools.partial(_argmin_body, n_keys),
        grid=(grid,),
        in_specs=[
            pl.BlockSpec((q, d), lambda i: (0, 0)),
            pl.BlockSpec((SB, d), lambda i: (2 * i, 0)),
            pl.BlockSpec((SB, d), lambda i: (2 * i + 1, 0)),
        ],
        out_specs=pl.BlockSpec((q, 1), lambda i: (0, 0)),
        out_shape=jax.ShapeDtypeStruct((q, 1), jnp.int32),
        scratch_shapes=[
            pltpu.VMEM((q, 1), jnp.float32),
            pltpu.VMEM((q, 1), jnp.int32),
        ],
    )(queries, keys, keys)


def _sc_chase_body(nearest_hbm, state_hbm, sidx_out, idx_v, sidx_v, sem):
    wid = lax.axis_index("s") * 2 + lax.axis_index("c")

    @pl.when(wid == 0)
    def _():
        pltpu.sync_copy(nearest_hbm, idx_v)
        pltpu.async_copy(state_hbm.at[idx_v], sidx_v, sem).wait()
        pltpu.sync_copy(sidx_v, sidx_out)


def _chase_sc(nearest, state_indices):
    q = nearest.shape[0]
    mesh = plsc.VectorSubcoreMesh(core_axis_name="c", subcore_axis_name="s")
    return pl.kernel(
        _sc_chase_body,
        out_type=jax.ShapeDtypeStruct((q,), jnp.int32),
        mesh=mesh,
        scratch_types=[
            pltpu.VMEM((q,), jnp.int32),
            pltpu.VMEM((q,), jnp.int32),
            pltpu.SemaphoreType.DMA,
        ],
    )(nearest, state_indices)


def _row_gather_body(sidx_ref, cc_ref, out_ref):
    i = pl.program_id(0)
    r = sidx_ref[i] % 8
    out_ref[pl.ds(i % 8, 1), :] = cc_ref[pl.ds(r, 1), :]


def _gather_tc(cluster_centers, sidx):
    q = sidx.shape[0]
    dc = cluster_centers.shape[1]
    grid_spec = pltpu.PrefetchScalarGridSpec(
        num_scalar_prefetch=1,
        grid=(q,),
        in_specs=[pl.BlockSpec((8, dc), lambda i, s: (s[i] // 8, 0))],
        out_specs=pl.BlockSpec((8, dc), lambda i, s: (i // 8, 0)),
    )
    return pl.pallas_call(
        _row_gather_body,
        grid_spec=grid_spec,
        out_shape=jax.ShapeDtypeStruct((q, dc), jnp.float32),
    )(sidx, cluster_centers)


def kernel(queries, keys, cluster_centers, state_indices):
    q = queries.shape[0]
    nearest = _nearest_tc(queries * -2.0, keys).reshape((q,))
    sidx = _chase_sc(nearest, state_indices)
    return _gather_tc(cluster_centers, sidx)
